# baseline (speedup 1.0000x reference)
import functools

import jax
import jax.numpy as jnp
from jax import lax
from jax.experimental import pallas as pl
from jax.experimental.pallas import tpu as pltpu
from jax.experimental.pallas import tpu_sc as plsc

L = 16
MAX_NUM = 5
IOU_THRESH = 0.5
K = 24
KTOT = K + MAX_NUM
BIG = 3.0e38


def _splat(x, dtype=None):
    v = lax.broadcast(x, (L,))
    return v if dtype is None else v.astype(dtype)


def _sc_body(nchunks, ngroups, boxes_hbm, s0_hbm, s1_hbm, out_hbm,
             s0_v, s1_v, bx_v, dv, cm, cm2, idx, bscr, outs, sem, sem2):
    s_idx = lax.axis_index("s")

    @pl.when(s_idx >= 0)
    def _():
        img = s_idx
        iota = lax.iota(jnp.int32, L)
        lane0 = iota == 0

        s0_cp = pltpu.async_copy(s0_hbm.at[img], s0_v, sem2)
        s1_cp = pltpu.async_copy(s1_hbm.at[img], s1_v, sem2)
        box_cp = pltpu.async_copy(boxes_hbm.at[img], bx_v, sem)
        s0_cp.wait()
        s1_cp.wait()

        def _build(g, _):
            base = g * L * L
            for u in range(L):
                sl = pl.ds(base + u * L, L)
                dv[sl] = s1_v[sl] - s0_v[sl]
            iv = _splat(base) + iota * L
            gm = plsc.load_gather(dv, [iv])
            for u in range(1, L):
                gm = jnp.maximum(gm, plsc.load_gather(dv, [iv + u]))
            cm[pl.ds(g * L, L)] = gm
            plsc.store_scatter(cm2, [_splat(g)], _splat(jnp.max(gm)),
                               mask=lane0)
            return _
        cm2[pl.ds(0, L)] = jnp.full((L,), -BIG, jnp.float32)
        cm2[pl.ds(L, L)] = jnp.full((L,), -BIG, jnp.float32)
        lax.fori_loop(0, ngroups, _build, None)

        idx[pl.ds(0, L)] = jnp.zeros((L,), jnp.int32)
        idx[pl.ds(L, L)] = jnp.zeros((L,), jnp.int32)

        def extract(k, carry):
            c2a, c2b = carry
            g = jnp.max(jnp.maximum(c2a, c2b))
            fa = plsc.all_reduce_ffs(c2a == g)
            fb = plsc.all_reduce_ffs(c2b == g)
            in_a = fa < L
            vstar = jnp.where(in_a, fa, fb + L)
            cmrow = plsc.load_gather(cm, [vstar * L + iota])
            lr = plsc.all_reduce_ffs(cmrow == g)
            cstar = vstar * L + lr
            dchunk = plsc.load_gather(dv, [cstar * L + iota])
            ld = plsc.all_reduce_ffs(dchunk == g)
            gidx = cstar * L + ld

            plsc.store_scatter(idx, [_splat(k)], gidx, mask=lane0)
            plsc.store_scatter(dv, [gidx], _splat(-BIG), mask=lane0)
            nm = jnp.max(jnp.where(iota == ld, -BIG, dchunk))
            plsc.store_scatter(cm, [cstar], _splat(nm), mask=lane0)
            rmp = jnp.max(jnp.where(iota == lr, -BIG, cmrow))
            rm = jnp.maximum(rmp, nm)
            c2a = jnp.where((iota == vstar) & in_a, rm, c2a)
            c2b = jnp.where((iota == vstar - L) & (~in_a), rm, c2b)
            return c2a, c2b
        lax.fori_loop(0, KTOT - 1, extract,
                      (cm2[pl.ds(0, L)], cm2[pl.ds(L, L)]))

        box_cp.wait()
        ia = idx[pl.ds(0, L)]
        ib = idx[pl.ds(L, L)]
        Xa, Xb = [], []
        for ci in range(4):
            civ = _splat(ci)
            xa = plsc.load_gather(bx_v, [civ, ia])
            xb = plsc.load_gather(bx_v, [civ, ib])
            bscr[ci, pl.ds(0, L)] = xa
            bscr[ci, pl.ds(L, L)] = xb
            Xa.append(xa)
            Xb.append(xb)

        area_a = (Xa[2] - Xa[0]) * (Xa[3] - Xa[1])
        area_b = (Xb[2] - Xb[0]) * (Xb[3] - Xb[1])
        valid_a = jnp.full((L,), True)
        valid_b = iota < (K - L)
        exv = jnp.full((L,), False)
        cur = [plsc.load_gather(bscr, [_splat(ci), _splat(K)])
               for ci in range(4)]

        for j in range(MAX_NUM - 1):
            fa = plsc.all_reduce_ffs(valid_a)
            fb = plsc.all_reduce_ffs(valid_b)
            fidx = jnp.where(fa < L, fa,
                             jnp.where(fb < L, fb + L, _splat(0)))
            mb = [jnp.where(exv, cur[ci],
                            plsc.load_gather(bscr, [_splat(ci), fidx]))
                  for ci in range(4)]
            a1 = (mb[2] - mb[0]) * (mb[3] - mb[1])

            iw_a = jnp.maximum(jnp.minimum(mb[2], Xa[2])
                               - jnp.maximum(mb[0], Xa[0]), 0.0)
            ih_a = jnp.maximum(jnp.minimum(mb[3], Xa[3])
                               - jnp.maximum(mb[1], Xa[1]), 0.0)
            inter_a = iw_a * ih_a
            iou_a = inter_a / (a1 + area_a - inter_a)
            iw_b = jnp.maximum(jnp.minimum(mb[2], Xb[2])
                               - jnp.maximum(mb[0], Xb[0]), 0.0)
            ih_b = jnp.maximum(jnp.minimum(mb[3], Xb[3])
                               - jnp.maximum(mb[1], Xb[1]), 0.0)
            inter_b = iw_b * ih_b
            iou_b = inter_b / (a1 + area_b - inter_b)

            over_a = valid_a & (iou_a >= IOU_THRESH) & (~exv)
            over_b = valid_b & (iou_b >= IOU_THRESH) & (~exv)
            x1m = jnp.minimum(jnp.min(jnp.where(over_a, Xa[0], BIG)),
                              jnp.min(jnp.where(over_b, Xb[0], BIG)))
            y1m = jnp.minimum(jnp.min(jnp.where(over_a, Xa[1], BIG)),
                              jnp.min(jnp.where(over_b, Xb[1], BIG)))
            x2m = jnp.maximum(jnp.max(jnp.where(over_a, Xa[2], -BIG)),
                              jnp.max(jnp.where(over_b, Xb[2], -BIG)))
            y2m = jnp.maximum(jnp.max(jnp.where(over_a, Xa[3], -BIG)),
                              jnp.max(jnp.where(over_b, Xb[3], -BIG)))
            roi = [x1m, y1m, x2m, y2m]
            for ci in range(4):
                val = jnp.where(exv, cur[ci], _splat(roi[ci]))
                plsc.store_scatter(outs, [_splat(j), _splat(ci)], val,
                                   mask=lane0)

            next_a = valid_a & (iou_a < IOU_THRESH)
            next_b = valid_b & (iou_b < IOU_THRESH)
            any_next = jnp.any(next_a) | jnp.any(next_b)
            newly = (~exv) & (~_splat(any_next))
            pick = exv | newly
            for ci in range(4):
                ph = plsc.load_gather(bscr, [_splat(ci), _splat(K + j)])
                cur[ci] = jnp.where(pick, ph, cur[ci])
            exv = exv | newly
            valid_a = next_a & (~exv)
            valid_b = next_b & (~exv)

        for ci in range(4):
            last = plsc.load_gather(bscr, [_splat(ci), _splat(KTOT - 2)])
            plsc.store_scatter(outs, [_splat(MAX_NUM - 1), _splat(ci)], last,
                               mask=lane0)
        pltpu.sync_copy(outs.at[pl.ds(0, MAX_NUM)], out_hbm.at[img])


def kernel(boxes, scores):
    B, N, _ = scores.shape
    npad = -N % 256
    NP = N + npad
    nchunks = NP // L
    ngroups = nchunks // L

    s0p = jnp.pad(scores[..., 0], ((0, 0), (0, npad)))
    s1p = jnp.pad(scores[..., 1], ((0, 0), (0, npad)), constant_values=-BIG)
    boxes_t = jnp.pad(jnp.transpose(boxes, (0, 2, 1)),
                      ((0, 0), (0, 0), (0, npad)))

    mesh = plsc.VectorSubcoreMesh(core_axis_name="c", subcore_axis_name="s",
                                  num_cores=1)
    body = functools.partial(_sc_body, nchunks, ngroups)
    out = pl.kernel(
        body,
        out_type=jax.ShapeDtypeStruct((B, MAX_NUM, 4), jnp.float32),
        mesh=mesh,
        compiler_params=pltpu.CompilerParams(needs_layout_passes=False),
        scratch_types=[
            pltpu.VMEM((NP,), jnp.float32),
            pltpu.VMEM((NP,), jnp.float32),
            pltpu.VMEM((4, NP), jnp.float32),
            pltpu.VMEM((NP,), jnp.float32),
            pltpu.VMEM((nchunks,), jnp.float32),
            pltpu.VMEM((2 * L,), jnp.float32),
            pltpu.VMEM((2 * L,), jnp.int32),
            pltpu.VMEM((4, 2 * L), jnp.float32),
            pltpu.VMEM((8, 4), jnp.float32),
            pltpu.SemaphoreType.DMA,
            pltpu.SemaphoreType.DMA,
        ],
    )(boxes_t, s0p, s1p)
    return out


# speedup vs baseline: 1.0248x; 1.0248x over previous
import functools

import jax
import jax.numpy as jnp
from jax import lax
from jax.experimental import pallas as pl
from jax.experimental.pallas import tpu as pltpu
from jax.experimental.pallas import tpu_sc as plsc

L = 16
MAX_NUM = 5
IOU_THRESH = 0.5
K = 24
KTOT = K + MAX_NUM
BIG = 3.0e38


def _splat(x, dtype=None):
    v = lax.broadcast(x, (L,))
    return v if dtype is None else v.astype(dtype)


def _sc_body(nchunks, ngroups, boxes_hbm, s0_hbm, s1_hbm, out_hbm,
             s0_v, s1_v, bx_v, dv, cm, cm2, idx, bscr, outs, sem, sem2):
    s_idx = lax.axis_index("s")

    @pl.when(s_idx >= 0)
    def _():
        img = s_idx
        iota = lax.iota(jnp.int32, L)
        lane0 = iota == 0

        s0_cp = pltpu.async_copy(s0_hbm.at[img], s0_v, sem2)
        s1_cp = pltpu.async_copy(s1_hbm.at[img], s1_v, sem2)
        box_cp = pltpu.async_copy(boxes_hbm.at[img], bx_v, sem)
        s0_cp.wait()
        s1_cp.wait()

        def _build(g, _):
            base = g * L * L
            ms = []
            for u in range(L):
                sl = pl.ds(base + u * L, L)
                d = s1_v[sl] - s0_v[sl]
                dv[sl] = d
                ms.append(jnp.max(d))
            gm = _splat(ms[0])
            for u in range(1, L):
                gm = jnp.where(iota == u, ms[u], gm)
            cm[pl.ds(g * L, L)] = gm
            plsc.store_scatter(cm2, [_splat(g)], _splat(jnp.max(gm)),
                               mask=lane0)
            return _
        cm2[pl.ds(0, L)] = jnp.full((L,), -BIG, jnp.float32)
        cm2[pl.ds(L, L)] = jnp.full((L,), -BIG, jnp.float32)
        lax.fori_loop(0, ngroups, _build, None)

        idx[pl.ds(0, L)] = jnp.zeros((L,), jnp.int32)
        idx[pl.ds(L, L)] = jnp.zeros((L,), jnp.int32)

        def extract(k, carry):
            c2a, c2b = carry
            g = jnp.max(jnp.maximum(c2a, c2b))
            fa = plsc.all_reduce_ffs(c2a == g)
            fb = plsc.all_reduce_ffs(c2b == g)
            in_a = fa < L
            vstar = jnp.where(in_a, fa, fb + L)
            cmrow = plsc.load_gather(cm, [vstar * L + iota])
            lr = plsc.all_reduce_ffs(cmrow == g)
            cstar = vstar * L + lr
            dchunk = plsc.load_gather(dv, [cstar * L + iota])
            ld = plsc.all_reduce_ffs(dchunk == g)
            gidx = cstar * L + ld

            plsc.store_scatter(idx, [_splat(k)], gidx, mask=lane0)
            plsc.store_scatter(dv, [gidx], _splat(-BIG), mask=lane0)
            nm = jnp.max(jnp.where(iota == ld, -BIG, dchunk))
            plsc.store_scatter(cm, [cstar], _splat(nm), mask=lane0)
            rmp = jnp.max(jnp.where(iota == lr, -BIG, cmrow))
            rm = jnp.maximum(rmp, nm)
            c2a = jnp.where((iota == vstar) & in_a, rm, c2a)
            c2b = jnp.where((iota == vstar - L) & (~in_a), rm, c2b)
            return c2a, c2b
        lax.fori_loop(0, KTOT - 1, extract,
                      (cm2[pl.ds(0, L)], cm2[pl.ds(L, L)]))

        box_cp.wait()
        ia = idx[pl.ds(0, L)]
        ib = idx[pl.ds(L, L)]
        Xa, Xb = [], []
        for ci in range(4):
            civ = _splat(ci)
            xa = plsc.load_gather(bx_v, [civ, ia])
            xb = plsc.load_gather(bx_v, [civ, ib])
            bscr[ci, pl.ds(0, L)] = xa
            bscr[ci, pl.ds(L, L)] = xb
            Xa.append(xa)
            Xb.append(xb)

        area_a = (Xa[2] - Xa[0]) * (Xa[3] - Xa[1])
        area_b = (Xb[2] - Xb[0]) * (Xb[3] - Xb[1])
        valid_a = jnp.full((L,), True)
        valid_b = iota < (K - L)
        exv = jnp.full((L,), False)
        cur = [plsc.load_gather(bscr, [_splat(ci), _splat(K)])
               for ci in range(4)]

        for j in range(MAX_NUM - 1):
            fa = plsc.all_reduce_ffs(valid_a)
            fb = plsc.all_reduce_ffs(valid_b)
            fidx = jnp.where(fa < L, fa,
                             jnp.where(fb < L, fb + L, _splat(0)))
            mb = [jnp.where(exv, cur[ci],
                            plsc.load_gather(bscr, [_splat(ci), fidx]))
                  for ci in range(4)]
            a1 = (mb[2] - mb[0]) * (mb[3] - mb[1])

            iw_a = jnp.maximum(jnp.minimum(mb[2], Xa[2])
                               - jnp.maximum(mb[0], Xa[0]), 0.0)
            ih_a = jnp.maximum(jnp.minimum(mb[3], Xa[3])
                               - jnp.maximum(mb[1], Xa[1]), 0.0)
            inter_a = iw_a * ih_a
            iou_a = inter_a / (a1 + area_a - inter_a)
            iw_b = jnp.maximum(jnp.minimum(mb[2], Xb[2])
                               - jnp.maximum(mb[0], Xb[0]), 0.0)
            ih_b = jnp.maximum(jnp.minimum(mb[3], Xb[3])
                               - jnp.maximum(mb[1], Xb[1]), 0.0)
            inter_b = iw_b * ih_b
            iou_b = inter_b / (a1 + area_b - inter_b)

            over_a = valid_a & (iou_a >= IOU_THRESH) & (~exv)
            over_b = valid_b & (iou_b >= IOU_THRESH) & (~exv)
            x1m = jnp.minimum(jnp.min(jnp.where(over_a, Xa[0], BIG)),
                              jnp.min(jnp.where(over_b, Xb[0], BIG)))
            y1m = jnp.minimum(jnp.min(jnp.where(over_a, Xa[1], BIG)),
                              jnp.min(jnp.where(over_b, Xb[1], BIG)))
            x2m = jnp.maximum(jnp.max(jnp.where(over_a, Xa[2], -BIG)),
                              jnp.max(jnp.where(over_b, Xb[2], -BIG)))
            y2m = jnp.maximum(jnp.max(jnp.where(over_a, Xa[3], -BIG)),
                              jnp.max(jnp.where(over_b, Xb[3], -BIG)))
            roi = [x1m, y1m, x2m, y2m]
            for ci in range(4):
                val = jnp.where(exv, cur[ci], _splat(roi[ci]))
                plsc.store_scatter(outs, [_splat(j), _splat(ci)], val,
                                   mask=lane0)

            next_a = valid_a & (iou_a < IOU_THRESH)
            next_b = valid_b & (iou_b < IOU_THRESH)
            any_next = jnp.any(next_a) | jnp.any(next_b)
            newly = (~exv) & (~_splat(any_next))
            pick = exv | newly
            for ci in range(4):
                ph = plsc.load_gather(bscr, [_splat(ci), _splat(K + j)])
                cur[ci] = jnp.where(pick, ph, cur[ci])
            exv = exv | newly
            valid_a = next_a & (~exv)
            valid_b = next_b & (~exv)

        for ci in range(4):
            last = plsc.load_gather(bscr, [_splat(ci), _splat(KTOT - 2)])
            plsc.store_scatter(outs, [_splat(MAX_NUM - 1), _splat(ci)], last,
                               mask=lane0)
        pltpu.sync_copy(outs.at[pl.ds(0, MAX_NUM)], out_hbm.at[img])


def kernel(boxes, scores):
    B, N, _ = scores.shape
    npad = -N % 256
    NP = N + npad
    nchunks = NP // L
    ngroups = nchunks // L

    s0p = jnp.pad(scores[..., 0], ((0, 0), (0, npad)))
    s1p = jnp.pad(scores[..., 1], ((0, 0), (0, npad)), constant_values=-BIG)
    boxes_t = jnp.pad(jnp.transpose(boxes, (0, 2, 1)),
                      ((0, 0), (0, 0), (0, npad)))

    mesh = plsc.VectorSubcoreMesh(core_axis_name="c", subcore_axis_name="s",
                                  num_cores=1)
    body = functools.partial(_sc_body, nchunks, ngroups)
    out = pl.kernel(
        body,
        out_type=jax.ShapeDtypeStruct((B, MAX_NUM, 4), jnp.float32),
        mesh=mesh,
        compiler_params=pltpu.CompilerParams(needs_layout_passes=False),
        scratch_types=[
            pltpu.VMEM((NP,), jnp.float32),
            pltpu.VMEM((NP,), jnp.float32),
            pltpu.VMEM((4, NP), jnp.float32),
            pltpu.VMEM((NP,), jnp.float32),
            pltpu.VMEM((nchunks,), jnp.float32),
            pltpu.VMEM((2 * L,), jnp.float32),
            pltpu.VMEM((2 * L,), jnp.int32),
            pltpu.VMEM((4, 2 * L), jnp.float32),
            pltpu.VMEM((8, 4), jnp.float32),
            pltpu.SemaphoreType.DMA,
            pltpu.SemaphoreType.DMA,
        ],
    )(boxes_t, s0p, s1p)
    return out


# speedup vs baseline: 1.0273x; 1.0025x over previous
import functools

import jax
import jax.numpy as jnp
from jax import lax
from jax.experimental import pallas as pl
from jax.experimental.pallas import tpu as pltpu
from jax.experimental.pallas import tpu_sc as plsc

L = 16
MAX_NUM = 5
IOU_THRESH = 0.5
K = 24
KTOT = K + MAX_NUM
BIG = 3.0e38


def _splat(x, dtype=None):
    v = lax.broadcast(x, (L,))
    return v if dtype is None else v.astype(dtype)


def _sc_body(nchunks, ngroups, boxes_hbm, s0_hbm, s1_hbm, out_hbm,
             s0_v, s1_v, bx_v, cm, cm2, idx, bscr, outs, sem, sem2):
    s_idx = lax.axis_index("s")

    @pl.when(s_idx >= 0)
    def _():
        img = s_idx
        iota = lax.iota(jnp.int32, L)
        lane0 = iota == 0

        s0_cp = pltpu.async_copy(s0_hbm.at[img], s0_v, sem2)
        s1_cp = pltpu.async_copy(s1_hbm.at[img], s1_v, sem2)
        box_cp = pltpu.async_copy(boxes_hbm.at[img], bx_v, sem)
        s0_cp.wait()
        s1_cp.wait()

        def _build(g, _):
            base = g * L * L
            ms = []
            for u in range(L):
                sl = pl.ds(base + u * L, L)
                d = s1_v[sl] - s0_v[sl]
                s0_v[sl] = d
                ms.append(jnp.max(d))
            gm = _splat(ms[0])
            for u in range(1, L):
                gm = jnp.where(iota == u, ms[u], gm)
            cm[pl.ds(g * L, L)] = gm
            plsc.store_scatter(cm2, [_splat(g)], _splat(jnp.max(gm)),
                               mask=lane0)
            return _
        cm2[pl.ds(0, L)] = jnp.full((L,), -BIG, jnp.float32)
        cm2[pl.ds(L, L)] = jnp.full((L,), -BIG, jnp.float32)
        lax.fori_loop(0, ngroups, _build, None)

        idx[pl.ds(0, L)] = jnp.zeros((L,), jnp.int32)
        idx[pl.ds(L, L)] = jnp.zeros((L,), jnp.int32)

        def extract(k, carry):
            c2a, c2b = carry
            g = jnp.max(jnp.maximum(c2a, c2b))
            fa = plsc.all_reduce_ffs(c2a == g)
            fb = plsc.all_reduce_ffs(c2b == g)
            in_a = fa < L
            vstar = jnp.where(in_a, fa, fb + L)
            cmrow = plsc.load_gather(cm, [vstar * L + iota])
            lr = plsc.all_reduce_ffs(cmrow == g)
            cstar = vstar * L + lr
            dchunk = plsc.load_gather(s0_v, [cstar * L + iota])
            ld = plsc.all_reduce_ffs(dchunk == g)
            gidx = cstar * L + ld

            plsc.store_scatter(idx, [_splat(k)], gidx, mask=lane0)
            plsc.store_scatter(s0_v, [gidx], _splat(-BIG), mask=lane0)
            nm = jnp.max(jnp.where(iota == ld, -BIG, dchunk))
            plsc.store_scatter(cm, [cstar], _splat(nm), mask=lane0)
            rmp = jnp.max(jnp.where(iota == lr, -BIG, cmrow))
            rm = jnp.maximum(rmp, nm)
            c2a = jnp.where((iota == vstar) & in_a, rm, c2a)
            c2b = jnp.where((iota == vstar - L) & (~in_a), rm, c2b)
            return c2a, c2b
        lax.fori_loop(0, KTOT - 1, extract,
                      (cm2[pl.ds(0, L)], cm2[pl.ds(L, L)]))

        box_cp.wait()
        ia = idx[pl.ds(0, L)]
        ib = idx[pl.ds(L, L)]
        Xa, Xb = [], []
        for ci in range(4):
            civ = _splat(ci)
            xa = plsc.load_gather(bx_v, [civ, ia])
            xb = plsc.load_gather(bx_v, [civ, ib])
            bscr[ci, pl.ds(0, L)] = xa
            bscr[ci, pl.ds(L, L)] = xb
            Xa.append(xa)
            Xb.append(xb)

        area_a = (Xa[2] - Xa[0]) * (Xa[3] - Xa[1])
        area_b = (Xb[2] - Xb[0]) * (Xb[3] - Xb[1])
        valid_a = jnp.full((L,), True)
        valid_b = iota < (K - L)
        exv = jnp.full((L,), False)
        cur = [plsc.load_gather(bscr, [_splat(ci), _splat(K)])
               for ci in range(4)]

        for j in range(MAX_NUM - 1):
            fa = plsc.all_reduce_ffs(valid_a)
            fb = plsc.all_reduce_ffs(valid_b)
            fidx = jnp.where(fa < L, fa,
                             jnp.where(fb < L, fb + L, _splat(0)))
            mb = [jnp.where(exv, cur[ci],
                            plsc.load_gather(bscr, [_splat(ci), fidx]))
                  for ci in range(4)]
            a1 = (mb[2] - mb[0]) * (mb[3] - mb[1])

            iw_a = jnp.maximum(jnp.minimum(mb[2], Xa[2])
                               - jnp.maximum(mb[0], Xa[0]), 0.0)
            ih_a = jnp.maximum(jnp.minimum(mb[3], Xa[3])
                               - jnp.maximum(mb[1], Xa[1]), 0.0)
            inter_a = iw_a * ih_a
            iou_a = inter_a / (a1 + area_a - inter_a)
            iw_b = jnp.maximum(jnp.minimum(mb[2], Xb[2])
                               - jnp.maximum(mb[0], Xb[0]), 0.0)
            ih_b = jnp.maximum(jnp.minimum(mb[3], Xb[3])
                               - jnp.maximum(mb[1], Xb[1]), 0.0)
            inter_b = iw_b * ih_b
            iou_b = inter_b / (a1 + area_b - inter_b)

            over_a = valid_a & (iou_a >= IOU_THRESH) & (~exv)
            over_b = valid_b & (iou_b >= IOU_THRESH) & (~exv)
            x1m = jnp.minimum(jnp.min(jnp.where(over_a, Xa[0], BIG)),
                              jnp.min(jnp.where(over_b, Xb[0], BIG)))
            y1m = jnp.minimum(jnp.min(jnp.where(over_a, Xa[1], BIG)),
                              jnp.min(jnp.where(over_b, Xb[1], BIG)))
            x2m = jnp.maximum(jnp.max(jnp.where(over_a, Xa[2], -BIG)),
                              jnp.max(jnp.where(over_b, Xb[2], -BIG)))
            y2m = jnp.maximum(jnp.max(jnp.where(over_a, Xa[3], -BIG)),
                              jnp.max(jnp.where(over_b, Xb[3], -BIG)))
            roi = [x1m, y1m, x2m, y2m]
            for ci in range(4):
                val = jnp.where(exv, cur[ci], _splat(roi[ci]))
                plsc.store_scatter(outs, [_splat(j), _splat(ci)], val,
                                   mask=lane0)

            next_a = valid_a & (iou_a < IOU_THRESH)
            next_b = valid_b & (iou_b < IOU_THRESH)
            any_next = jnp.any(next_a) | jnp.any(next_b)
            newly = (~exv) & (~_splat(any_next))
            pick = exv | newly
            for ci in range(4):
                ph = plsc.load_gather(bscr, [_splat(ci), _splat(K + j)])
                cur[ci] = jnp.where(pick, ph, cur[ci])
            exv = exv | newly
            valid_a = next_a & (~exv)
            valid_b = next_b & (~exv)

        for ci in range(4):
            last = plsc.load_gather(bscr, [_splat(ci), _splat(KTOT - 2)])
            plsc.store_scatter(outs, [_splat(MAX_NUM - 1), _splat(ci)], last,
                               mask=lane0)
        pltpu.sync_copy(outs.at[pl.ds(0, MAX_NUM)], out_hbm.at[img])


def kernel(boxes, scores):
    B, N, _ = scores.shape
    npad = -N % 256
    NP = N + npad
    nchunks = NP // L
    ngroups = nchunks // L

    s0p = jnp.pad(scores[..., 0], ((0, 0), (0, npad)))
    s1p = jnp.pad(scores[..., 1], ((0, 0), (0, npad)), constant_values=-BIG)
    boxes_t = jnp.pad(jnp.transpose(boxes, (0, 2, 1)),
                      ((0, 0), (0, 0), (0, npad)))

    mesh = plsc.VectorSubcoreMesh(core_axis_name="c", subcore_axis_name="s",
                                  num_cores=1)
    body = functools.partial(_sc_body, nchunks, ngroups)
    out = pl.kernel(
        body,
        out_type=jax.ShapeDtypeStruct((B, MAX_NUM, 4), jnp.float32),
        mesh=mesh,
        compiler_params=pltpu.CompilerParams(needs_layout_passes=False),
        scratch_types=[
            pltpu.VMEM((NP,), jnp.float32),
            pltpu.VMEM((NP,), jnp.float32),
            pltpu.VMEM((4, NP), jnp.float32),
            pltpu.VMEM((nchunks,), jnp.float32),
            pltpu.VMEM((2 * L,), jnp.float32),
            pltpu.VMEM((2 * L,), jnp.int32),
            pltpu.VMEM((4, 2 * L), jnp.float32),
            pltpu.VMEM((8, 4), jnp.float32),
            pltpu.SemaphoreType.DMA,
            pltpu.SemaphoreType.DMA,
        ],
    )(boxes_t, s0p, s1p)
    return out
